# baseline (speedup 1.0000x reference)
import functools

import jax
import jax.numpy as jnp
from jax.experimental import pallas as pl
from jax.experimental.pallas import tpu as pltpu

_TOP_K = 8
_EXPERTS = 64
_BLOCK_ROWS = 2048
_BLOCK_D = 1024


def _gate_kernel(x_ref, w_ref, idx_ref, val_ref, acc_ref):
    j = pl.program_id(1)
    nj = pl.num_programs(1)
    partial = jax.lax.dot_general(
        w_ref[...], x_ref[...], (((1,), (1,)), ((), ())),
        preferred_element_type=jnp.float32,
    )

    @pl.when(j == 0)
    def _init():
        acc_ref[...] = partial

    @pl.when(j > 0)
    def _accum():
        acc_ref[...] += partial

    @pl.when(j == nj - 1)
    def _epilogue():
        logits = acc_ref[...]
        b = logits.view(jnp.int32)
        key = b ^ ((b >> 31) & jnp.int32(0x7FFFFFFF))
        iota = jax.lax.broadcasted_iota(jnp.int32, logits.shape, 0)
        key = (key & jnp.int32(~63)) | (jnp.int32(63) - iota)
        cols = []
        for _ in range(_TOP_K):
            t = jnp.maximum(key[:32], key[32:])
            t = jnp.maximum(t[:16], t[16:])
            t = jnp.maximum(t[:8], t[8:])
            m = jnp.max(t, axis=0, keepdims=True)
            cols.append(m)
            key = jnp.where(key == m, jnp.int32(-0x80000000), key)
        top = jnp.concatenate(cols, axis=0)
        topi = jnp.int32(63) - (top & jnp.int32(63))
        vb = (top | jnp.int32(32)) & jnp.int32(~31)
        vb = vb ^ ((vb >> 31) & jnp.int32(0x7FFFFFFF))
        topv = vb.view(jnp.float32)
        e = jnp.exp(topv - topv[:1])
        val_ref[...] = e / jnp.sum(e, axis=0, keepdims=True)
        idx_ref[...] = topi


@functools.partial(jax.jit, static_argnames=())
def kernel(hidden_states, weight):
    bsz, seq_len, h = hidden_states.shape
    n = bsz * seq_len
    x = hidden_states.reshape(n, h)
    grid = (n // _BLOCK_ROWS, h // _BLOCK_D)
    idx_t, val_t = pl.pallas_call(
        _gate_kernel,
        grid=grid,
        in_specs=[
            pl.BlockSpec((_BLOCK_ROWS, _BLOCK_D), lambda i, j: (i, j)),
            pl.BlockSpec((_EXPERTS, _BLOCK_D), lambda i, j: (0, j)),
        ],
        out_specs=[
            pl.BlockSpec((_TOP_K, _BLOCK_ROWS), lambda i, j: (0, i)),
            pl.BlockSpec((_TOP_K, _BLOCK_ROWS), lambda i, j: (0, i)),
        ],
        out_shape=[
            jax.ShapeDtypeStruct((_TOP_K, n), jnp.int32),
            jax.ShapeDtypeStruct((_TOP_K, n), jnp.float32),
        ],
        scratch_shapes=[pltpu.VMEM((_EXPERTS, _BLOCK_ROWS), jnp.float32)],
        compiler_params=pltpu.CompilerParams(
            dimension_semantics=("arbitrary", "arbitrary"),
        ),
    )(x, weight)
    return idx_t.T, val_t.T


# speedup vs baseline: 1.1050x; 1.1050x over previous
import functools

import jax
import jax.numpy as jnp
from jax.experimental import pallas as pl
from jax.experimental.pallas import tpu as pltpu

_TOP_K = 8
_EXPERTS = 64
_BLOCK_ROWS = 1024


def _gate_kernel(x_ref, w_ref, idx_ref, val_ref):
    x = x_ref[...]
    w = w_ref[...]
    logits = jax.lax.dot_general(
        w, x, (((1,), (1,)), ((), ())), preferred_element_type=jnp.float32
    )
    b = logits.view(jnp.int32)
    key = b ^ ((b >> 31) & jnp.int32(0x7FFFFFFF))
    iota = jax.lax.broadcasted_iota(jnp.int32, logits.shape, 0)
    key = (key & jnp.int32(~63)) | (jnp.int32(63) - iota)
    cols = []
    for _ in range(_TOP_K):
        t = jnp.maximum(key[:32], key[32:])
        t = jnp.maximum(t[:16], t[16:])
        t = jnp.maximum(t[:8], t[8:])
        m = jnp.max(t, axis=0, keepdims=True)
        cols.append(m)
        key = jnp.where(key == m, jnp.int32(-0x80000000), key)
    top = jnp.concatenate(cols, axis=0)
    topi = jnp.int32(63) - (top & jnp.int32(63))
    vb = (top | jnp.int32(32)) & jnp.int32(~31)
    vb = vb ^ ((vb >> 31) & jnp.int32(0x7FFFFFFF))
    topv = vb.view(jnp.float32)
    e = jnp.exp(topv - topv[:1])
    val_ref[...] = e / jnp.sum(e, axis=0, keepdims=True)
    idx_ref[...] = topi


@functools.partial(jax.jit, static_argnames=())
def kernel(hidden_states, weight):
    bsz, seq_len, h = hidden_states.shape
    n = bsz * seq_len
    x = hidden_states.reshape(n, h)
    grid = (n // _BLOCK_ROWS,)
    idx_t, val_t = pl.pallas_call(
        _gate_kernel,
        grid=grid,
        in_specs=[
            pl.BlockSpec((_BLOCK_ROWS, h), lambda i: (i, 0)),
            pl.BlockSpec((_EXPERTS, h), lambda i: (0, 0)),
        ],
        out_specs=[
            pl.BlockSpec((_TOP_K, _BLOCK_ROWS), lambda i: (0, i)),
            pl.BlockSpec((_TOP_K, _BLOCK_ROWS), lambda i: (0, i)),
        ],
        out_shape=[
            jax.ShapeDtypeStruct((_TOP_K, n), jnp.int32),
            jax.ShapeDtypeStruct((_TOP_K, n), jnp.float32),
        ],
        compiler_params=pltpu.CompilerParams(
            dimension_semantics=("arbitrary",),
        ),
    )(x, weight)
    return idx_t.T, val_t.T
